# baseline (speedup 1.0000x reference)
import functools

import jax
import jax.numpy as jnp
from jax import lax
from jax.experimental import pallas as pl
from jax.experimental.pallas import tpu as pltpu
from jax.experimental.pallas import tpu_sc as plsc

EMBED_DIM = 128
VOCAB_ROWS = 65
GPAD = 64


def _combine_tables(t1, t2):
    def body(t1_ref, t2_ref, out_ref):
        out_ref[...] = t1_ref[...][:, None, :] + t2_ref[...][None, :, :]

    out = pl.pallas_call(
        body,
        out_shape=jax.ShapeDtypeStruct(
            (VOCAB_ROWS, VOCAB_ROWS, EMBED_DIM), jnp.float32),
    )(t1, t2)
    return out.reshape(VOCAB_ROWS * VOCAB_ROWS, EMBED_DIM)


def _make_sc_kernel(n_batch: int, nbuf: int, num_workers: int):
    groups = n_batch // 2
    groups_per_w = groups // num_workers
    assert groups_per_w % nbuf == 0 and groups_per_w >= 2 * nbuf
    mesh = plsc.VectorSubcoreMesh(core_axis_name="c", subcore_axis_name="s")

    @functools.partial(
        pl.kernel,
        mesh=mesh,
        out_type=jax.ShapeDtypeStruct((n_batch * GPAD, EMBED_DIM),
                                      jnp.float32),
        scratch_types=[
            pltpu.VMEM((groups_per_w * 2 * GPAD,), jnp.int32),
            pltpu.VMEM((nbuf, 2 * GPAD, EMBED_DIM), jnp.float32),
        ]
        + [pltpu.SemaphoreType.DMA] * (2 * nbuf),
    )
    def sc_kernel(pidx_hbm, t12_hbm, out_hbm, pidx_v, rows_v, *sems):
        gsem = sems[:nbuf]
        osem = sems[nbuf:]
        wid = lax.axis_index("s") * 2 + lax.axis_index("c")
        g0 = wid * groups_per_w
        chunk = 2 * GPAD

        pltpu.sync_copy(pidx_hbm.at[pl.ds(g0 * chunk, groups_per_w * chunk)],
                        pidx_v)

        def gather(g, s):
            pltpu.async_copy(
                t12_hbm.at[pidx_v.at[pl.ds(g * chunk, chunk)]],
                rows_v.at[s], gsem[s])

        def gather_wait(s):
            pltpu.make_async_copy(
                t12_hbm.at[pl.ds(0, chunk)], rows_v.at[s], gsem[s]).wait()

        def out_start(g, s):
            pltpu.async_copy(
                rows_v.at[s],
                out_hbm.at[pl.ds((g0 + g) * chunk, chunk)], osem[s])

        def out_wait(s):
            pltpu.make_async_copy(
                rows_v.at[s], out_hbm.at[pl.ds(0, chunk)], osem[s]).wait()

        for s in range(nbuf):
            gather(s, s)

        def ring(i, _):
            p0 = i * nbuf
            for s in range(nbuf):
                g = p0 + s
                gather_wait(s)
                out_start(g, s)
                nxt = g + nbuf

                @pl.when(nxt < groups_per_w)
                def _():
                    out_wait(s)
                    gather(nxt, s)
            return 0

        lax.fori_loop(0, groups_per_w // nbuf, ring, 0, unroll=False)
        for s in range(nbuf):
            out_wait(s)

    return sc_kernel


def kernel(initial_position_indexes, destination_indexes,
           initial_position_table, destination_table):
    b, l = initial_position_indexes.shape
    p = initial_position_indexes.astype(jnp.int32) * VOCAB_ROWS \
        + destination_indexes.astype(jnp.int32)
    p2 = p.reshape(b // 2, 2 * l)
    pidx = jnp.zeros((b // 2, 2 * GPAD), jnp.int32)
    pidx = pidx.at[:, :l].set(p2[:, :l])
    pidx = pidx.at[:, GPAD:GPAD + l].set(p2[:, l:])
    pidx = pidx.reshape(b * GPAD)

    t12 = _combine_tables(initial_position_table, destination_table)
    sc = _make_sc_kernel(n_batch=b, nbuf=4, num_workers=32)
    out_pad = sc(pidx, t12)
    return out_pad.reshape(b, GPAD, EMBED_DIM)[:, :l, :]


# speedup vs baseline: 8.6164x; 8.6164x over previous
import functools

import jax
import jax.numpy as jnp
from jax import lax
from jax.experimental import pallas as pl
from jax.experimental.pallas import tpu as pltpu
from jax.experimental.pallas import tpu_sc as plsc

EMBED_DIM = 128
VOCAB_ROWS = 65
GPAD = 64


def _combine_tables(t1, t2):
    def body(t1_ref, t2_ref, out_ref):
        out_ref[...] = t1_ref[...][:, None, :] + t2_ref[...][None, :, :]

    out = pl.pallas_call(
        body,
        out_shape=jax.ShapeDtypeStruct(
            (VOCAB_ROWS, VOCAB_ROWS, EMBED_DIM), jnp.float32),
    )(t1, t2)
    return out.reshape(VOCAB_ROWS * VOCAB_ROWS, EMBED_DIM)


def _make_sc_kernel(n_batch: int, nbuf: int, num_workers: int):
    groups = n_batch // 2
    groups_per_w = groups // num_workers
    assert groups_per_w % nbuf == 0 and groups_per_w >= 2 * nbuf
    mesh = plsc.VectorSubcoreMesh(core_axis_name="c", subcore_axis_name="s")

    @functools.partial(
        pl.kernel,
        mesh=mesh,
        out_type=jax.ShapeDtypeStruct((n_batch * GPAD, EMBED_DIM),
                                      jnp.float32),
        scratch_types=[
            pltpu.VMEM((groups_per_w * 2 * GPAD,), jnp.int32),
            pltpu.VMEM((nbuf, 2 * GPAD, EMBED_DIM), jnp.float32),
        ]
        + [pltpu.SemaphoreType.DMA] * (2 * nbuf),
    )
    def sc_kernel(pidx_hbm, t12_hbm, out_hbm, pidx_v, rows_v, *sems):
        gsem = sems[:nbuf]
        osem = sems[nbuf:]
        wid = lax.axis_index("s") * 2 + lax.axis_index("c")
        g0 = wid * groups_per_w
        chunk = 2 * GPAD

        pltpu.sync_copy(pidx_hbm.at[pl.ds(g0 * chunk, groups_per_w * chunk)],
                        pidx_v)

        def gather(g, s):
            pltpu.async_copy(
                t12_hbm.at[pidx_v.at[pl.ds(g * chunk, chunk)]],
                rows_v.at[s], gsem[s])

        def gather_wait(s):
            pltpu.make_async_copy(
                t12_hbm.at[pl.ds(0, chunk)], rows_v.at[s], gsem[s]).wait()

        def out_start(g, s):
            pltpu.async_copy(
                rows_v.at[s],
                out_hbm.at[pl.ds((g0 + g) * chunk, chunk)], osem[s])

        def out_wait(s):
            pltpu.make_async_copy(
                rows_v.at[s], out_hbm.at[pl.ds(0, chunk)], osem[s]).wait()

        for s in range(nbuf):
            gather(s, s)

        def ring(i, _):
            p0 = i * nbuf
            for s in range(nbuf):
                g = p0 + s
                gather_wait(s)
                out_start(g, s)
                nxt = g + nbuf

                @pl.when(nxt < groups_per_w)
                def _():
                    out_wait(s)
                    gather(nxt, s)
            return 0

        lax.fori_loop(0, groups_per_w // nbuf, ring, 0, unroll=False)
        for s in range(nbuf):
            out_wait(s)

    return sc_kernel


def kernel(initial_position_indexes, destination_indexes,
           initial_position_table, destination_table):
    b, l = initial_position_indexes.shape
    p = initial_position_indexes.astype(jnp.int32) * VOCAB_ROWS \
        + destination_indexes.astype(jnp.int32)
    p2 = p.reshape(b // 2, 2 * l)
    pidx = (lax.broadcasted_iota(jnp.int32, (b // 2, 2 * GPAD), 0) * 131
            + lax.broadcasted_iota(jnp.int32, (b // 2, 2 * GPAD), 1)) \
        % (VOCAB_ROWS * VOCAB_ROWS)
    pidx = pidx.at[:, :l].set(p2[:, :l])
    pidx = pidx.at[:, GPAD:GPAD + l].set(p2[:, l:])
    pidx = pidx.reshape(b * GPAD)

    t12 = _combine_tables(initial_position_table, destination_table)
    sc = _make_sc_kernel(n_batch=b, nbuf=4, num_workers=32)
    out_pad = sc(pidx, t12)
    return out_pad.reshape(b, GPAD, EMBED_DIM)[:, :l, :]


# speedup vs baseline: 14.0790x; 1.6340x over previous
import functools

import jax
import jax.numpy as jnp
from jax import lax
from jax.experimental import pallas as pl
from jax.experimental.pallas import tpu as pltpu
from jax.experimental.pallas import tpu_sc as plsc

EMBED_DIM = 128
VOCAB_ROWS = 65
GPAD = 64


def _combine_tables(t1, t2):
    def body(t1_ref, t2_ref, out_ref):
        out_ref[...] = t1_ref[...][:, None, :] + t2_ref[...][None, :, :]

    out = pl.pallas_call(
        body,
        out_shape=jax.ShapeDtypeStruct(
            (VOCAB_ROWS, VOCAB_ROWS, EMBED_DIM), jnp.float32),
    )(t1, t2)
    return out.reshape(VOCAB_ROWS * VOCAB_ROWS, EMBED_DIM)


def _make_sc_kernel(n_batch: int, seq: int, nbuf: int, num_workers: int):
    rows_per_w = n_batch // num_workers
    pairs_per_w = rows_per_w // 2
    assert pairs_per_w % nbuf == 0 and pairs_per_w >= 2 * nbuf
    mesh = plsc.VectorSubcoreMesh(core_axis_name="c", subcore_axis_name="s")

    @functools.partial(
        pl.kernel,
        mesh=mesh,
        out_type=jax.ShapeDtypeStruct((n_batch, seq, EMBED_DIM), jnp.float32),
        scratch_types=[
            pltpu.VMEM((rows_per_w * GPAD,), jnp.int32),
            pltpu.VMEM((nbuf, 2 * GPAD, EMBED_DIM), jnp.float32),
        ]
        + [pltpu.SemaphoreType.DMA] * (2 * nbuf),
        compiler_params=pltpu.CompilerParams(use_tc_tiling_on_sc=True),
    )
    def sc_kernel(pidx_hbm, t12_hbm, out_hbm, pidx_v, rows_v, *sems):
        gsem = sems[:nbuf]
        osem = sems[nbuf:]
        wid = lax.axis_index("s") * 2 + lax.axis_index("c")
        b0 = wid * rows_per_w
        chunk = 2 * GPAD

        pltpu.sync_copy(pidx_hbm.at[pl.ds(b0 * GPAD, rows_per_w * GPAD)],
                        pidx_v)

        def gather(p, s):
            pltpu.async_copy(
                t12_hbm.at[pidx_v.at[pl.ds(p * chunk, chunk)]],
                rows_v.at[s], gsem[s])

        def gather_wait(s):
            pltpu.make_async_copy(
                t12_hbm.at[pl.ds(0, chunk)], rows_v.at[s], gsem[s]).wait()

        def out_start(p, s):
            b = b0 + 2 * p
            pltpu.async_copy(rows_v.at[s, pl.ds(0, seq)],
                             out_hbm.at[b], osem[s])
            pltpu.async_copy(rows_v.at[s, pl.ds(GPAD, seq)],
                             out_hbm.at[b + 1], osem[s])

        def out_wait(s):
            for _ in range(2):
                pltpu.make_async_copy(rows_v.at[s, pl.ds(0, seq)],
                                      out_hbm.at[b0], osem[s]).wait()

        for s in range(nbuf):
            gather(s, s)

        def ring(i, _):
            p0 = i * nbuf
            for s in range(nbuf):
                p = p0 + s
                gather_wait(s)
                out_start(p, s)
                nxt = p + nbuf

                @pl.when(nxt < pairs_per_w)
                def _():
                    out_wait(s)
                    gather(nxt, s)
            return 0

        lax.fori_loop(0, pairs_per_w // nbuf, ring, 0, unroll=False)
        for s in range(nbuf):
            out_wait(s)

    return sc_kernel


def kernel(initial_position_indexes, destination_indexes,
           initial_position_table, destination_table):
    b, l = initial_position_indexes.shape
    p = initial_position_indexes.astype(jnp.int32) * VOCAB_ROWS \
        + destination_indexes.astype(jnp.int32)
    pidx = (lax.broadcasted_iota(jnp.int32, (b, GPAD), 0) * 131
            + lax.broadcasted_iota(jnp.int32, (b, GPAD), 1) * 7) \
        % (VOCAB_ROWS * VOCAB_ROWS)
    pidx = pidx.at[:, :l].set(p).reshape(b * GPAD)

    t12 = _combine_tables(initial_position_table, destination_table)
    sc = _make_sc_kernel(n_batch=b, seq=l, nbuf=4, num_workers=32)
    return sc(pidx, t12)


# speedup vs baseline: 23.5236x; 1.6708x over previous
import functools

import jax
import jax.numpy as jnp
from jax import lax
from jax.experimental import pallas as pl
from jax.experimental.pallas import tpu as pltpu
from jax.experimental.pallas import tpu_sc as plsc

EMBED_DIM = 128
VOCAB_ROWS = 65
LSEQ_PAD = 56


def _tc_prep(i1, i2, t1, t2):
    b, l = i1.shape

    def body(i1_ref, i2_ref, t1_ref, t2_ref, pidx_ref, t12_ref):
        p = i1_ref[...] * VOCAB_ROWS + i2_ref[...]
        pidx_ref[:l, :] = p.T
        t12_ref[...] = t1_ref[...][:, None, :] + t2_ref[...][None, :, :]

    pidx, t12 = pl.pallas_call(
        body,
        out_shape=(
            jax.ShapeDtypeStruct((LSEQ_PAD, b), jnp.int32),
            jax.ShapeDtypeStruct((VOCAB_ROWS, VOCAB_ROWS, EMBED_DIM),
                                 jnp.float32),
        ),
    )(i1, i2, t1, t2)
    return (pidx.reshape(LSEQ_PAD * b),
            t12.reshape(VOCAB_ROWS * VOCAB_ROWS, EMBED_DIM))


def _make_sc_kernel(n_rows: int, chunk: int, nbuf: int, num_workers: int):
    per_w = n_rows // num_workers
    n_chunks = per_w // chunk
    assert n_chunks % nbuf == 0 and n_chunks >= 2 * nbuf
    mesh = plsc.VectorSubcoreMesh(core_axis_name="c", subcore_axis_name="s")

    @functools.partial(
        pl.kernel,
        mesh=mesh,
        out_type=jax.ShapeDtypeStruct((n_rows, EMBED_DIM), jnp.float32),
        scratch_types=[
            pltpu.VMEM((per_w,), jnp.int32),
            pltpu.VMEM((nbuf, chunk, EMBED_DIM), jnp.float32),
        ]
        + [pltpu.SemaphoreType.DMA] * (2 * nbuf),
    )
    def sc_kernel(pidx_hbm, t12_hbm, out_hbm, pidx_v, rows_v, *sems):
        gsem = sems[:nbuf]
        osem = sems[nbuf:]
        wid = lax.axis_index("s") * 2 + lax.axis_index("c")
        base = wid * per_w

        pltpu.sync_copy(pidx_hbm.at[pl.ds(base, per_w)], pidx_v)

        def gather(g, s):
            pltpu.async_copy(
                t12_hbm.at[pidx_v.at[pl.ds(g * chunk, chunk)]],
                rows_v.at[s], gsem[s])

        def gather_wait(s):
            pltpu.make_async_copy(
                t12_hbm.at[pl.ds(0, chunk)], rows_v.at[s], gsem[s]).wait()

        def out_start(g, s):
            pltpu.async_copy(
                rows_v.at[s],
                out_hbm.at[pl.ds(base + g * chunk, chunk)], osem[s])

        def out_wait(s):
            pltpu.make_async_copy(
                rows_v.at[s], out_hbm.at[pl.ds(0, chunk)], osem[s]).wait()

        for s in range(nbuf):
            gather(s, s)

        def ring(i, _):
            g0 = i * nbuf
            for s in range(nbuf):
                g = g0 + s
                gather_wait(s)
                out_start(g, s)
                nxt = g + nbuf

                @pl.when(nxt < n_chunks)
                def _():
                    out_wait(s)
                    gather(nxt, s)
            return 0

        lax.fori_loop(0, n_chunks // nbuf, ring, 0, unroll=False)
        for s in range(nbuf):
            out_wait(s)

    return sc_kernel


def kernel(initial_position_indexes, destination_indexes,
           initial_position_table, destination_table):
    b, l = initial_position_indexes.shape
    pidx, t12 = _tc_prep(
        initial_position_indexes.astype(jnp.int32),
        destination_indexes.astype(jnp.int32),
        initial_position_table, destination_table)
    sc = _make_sc_kernel(n_rows=l * b, chunk=128, nbuf=5, num_workers=32)
    out_flat = sc(pidx, t12)
    return out_flat.reshape(l, b, EMBED_DIM).transpose(1, 0, 2)
